# baseline (speedup 1.0000x reference)
import jax
import jax.numpy as jnp
from jax import lax
from jax.experimental import pallas as pl
from jax.experimental.pallas import tpu as pltpu
from jax.experimental.pallas import tpu_sc as plsc

N = 2_000_000
R = 128
C = 3200
CHUNKS = N // C
NW = 32
QCH = (157, 156, 156, 156)


def _make_body(nch, iters):
    def _body(x_hbm, y_hbm, z_hbm, b_hbm, tab_hbm, out_hbm,
              xv, yv, zv, bv, idxv, gv, sem_in, sem_g, sem_o):
        w = lax.axis_index("s") * 2 + lax.axis_index("c")

        def start_in(k, p):
            c = w + NW * k

            @pl.when(c < nch)
            def _():
                sl = pl.ds(c * C, C)
                pltpu.async_copy(x_hbm.at[sl], xv[p], sem_in[p])
                pltpu.async_copy(y_hbm.at[sl], yv[p], sem_in[p])
                pltpu.async_copy(z_hbm.at[sl], zv[p], sem_in[p])
                pltpu.async_copy(b_hbm.at[sl], bv[p], sem_in[p])

        def wait_in(k, p):
            c = w + NW * k

            @pl.when(c < nch)
            def _():
                sl = pl.ds(0, C)
                pltpu.make_async_copy(x_hbm.at[sl], xv[p], sem_in[p]).wait()
                pltpu.make_async_copy(y_hbm.at[sl], yv[p], sem_in[p]).wait()
                pltpu.make_async_copy(z_hbm.at[sl], zv[p], sem_in[p]).wait()
                pltpu.make_async_copy(b_hbm.at[sl], bv[p], sem_in[p]).wait()

        def compute(k, p):
            c = w + NW * k

            @pl.when(c < nch)
            def _():
                def step(i, carry):
                    s = pl.ds(i * 16, 16)
                    xf = xv[p][s]
                    yf = yv[p][s]
                    zf = zv[p][s]
                    bi = bv[p][s]
                    gx = ((xf * 0.5 + 0.5) * 128.0).astype(jnp.int32)
                    gy = ((yf * 0.5 + 0.5) * 128.0).astype(jnp.int32)
                    gz = ((zf * 0.5 + 0.5) * 128.0).astype(jnp.int32)
                    gx = jnp.minimum(jnp.maximum(gx, 0), 127)
                    gy = jnp.minimum(jnp.maximum(gy, 0), 127)
                    gz = jnp.minimum(jnp.maximum(gz, 0), 127)
                    f = ((bi * 128 + gx) * 128 + gy) * 128 + gz
                    idxv[p][s] = f
                    return carry

                lax.fori_loop(0, C // 16, step, 0)

        def fire_gather(k, p):
            c = w + NW * k

            @pl.when(c < nch)
            def _():
                pltpu.async_copy(tab_hbm.at[idxv[p]], gv[p], sem_g[p])

        def wait_out(k, p):
            c = w + NW * k

            @pl.when(c < nch)
            def _():
                pltpu.make_async_copy(gv[p], out_hbm.at[pl.ds(0, C)],
                                      sem_o[p]).wait()

        def drain_gather(k, p):
            c = w + NW * k

            @pl.when(c < nch)
            def _():
                pltpu.make_async_copy(tab_hbm.at[idxv[p]], gv[p],
                                      sem_g[p]).wait()
                pltpu.async_copy(gv[p], out_hbm.at[pl.ds(c * C, C)], sem_o[p])

        start_in(0, 0)

        def outer(it, carry):
            for p in (0, 1):
                k = 2 * it + p
                wait_in(k, p)
                compute(k, p)

                @pl.when(k > 1)
                def _():
                    wait_out(k - 2, p)

                fire_gather(k, p)

                @pl.when(k > 0)
                def _():
                    drain_gather(k - 1, p ^ 1)

                start_in(k + 1, p ^ 1)
            return carry

        lax.fori_loop(0, iters // 2, outer, 0)
        drain_gather(iters - 1, (iters - 1) & 1)
        wait_out(iters - 2, (iters - 2) & 1)
        wait_out(iters - 1, (iters - 1) & 1)

    return _body


def kernel(pts, bidx, occ_grid_per_batch, resolution):
    tab = occ_grid_per_batch.reshape(-1)
    x = pts[:, 0]
    y = pts[:, 1]
    z = pts[:, 2]

    mesh = plsc.VectorSubcoreMesh(core_axis_name="c", subcore_axis_name="s")
    outs = []
    off = 0
    for nch in QCH:
        npts = nch * C
        iters = -(-nch // NW)
        iters += iters & 1
        sl = slice(off, off + npts)
        out_q = pl.kernel(
            _make_body(nch, iters),
            out_type=jax.ShapeDtypeStruct((npts,), jnp.bool_),
            mesh=mesh,
            scratch_types=[
                [pltpu.VMEM((C,), jnp.float32)] * 2,
                [pltpu.VMEM((C,), jnp.float32)] * 2,
                [pltpu.VMEM((C,), jnp.float32)] * 2,
                [pltpu.VMEM((C,), jnp.int32)] * 2,
                [pltpu.VMEM((C,), jnp.int32)] * 2,
                [pltpu.VMEM((C,), jnp.bool_)] * 2,
                [pltpu.SemaphoreType.DMA] * 2,
                [pltpu.SemaphoreType.DMA] * 2,
                [pltpu.SemaphoreType.DMA] * 2,
            ],
        )(x[sl], y[sl], z[sl], bidx[sl], tab)
        outs.append(out_q)
        off += npts
    return jnp.concatenate(outs)


# speedup vs baseline: 1.3028x; 1.3028x over previous
import jax
import jax.numpy as jnp
from jax import lax
from jax.experimental import pallas as pl
from jax.experimental.pallas import tpu as pltpu
from jax.experimental.pallas import tpu_sc as plsc

N = 2_000_000
R = 128
C = 8000
CHUNKS = N // C
NW = 32
ITERS = (CHUNKS + NW - 1) // NW


def _body(x_hbm, y_hbm, z_hbm, b_hbm, tab_hbm, out_hbm,
          xv, yv, zv, bv, idxv, gv, sem_in, sem_g, sem_o):
    w = lax.axis_index("s") * 2 + lax.axis_index("c")

    def start_in(k, p):
        c = w + NW * k

        @pl.when(c < CHUNKS)
        def _():
            sl = pl.ds(c * C, C)
            pltpu.async_copy(x_hbm.at[sl], xv[p], sem_in[p])
            pltpu.async_copy(y_hbm.at[sl], yv[p], sem_in[p])
            pltpu.async_copy(z_hbm.at[sl], zv[p], sem_in[p])
            pltpu.async_copy(b_hbm.at[sl], bv[p], sem_in[p])

    def wait_in(k, p):
        c = w + NW * k

        @pl.when(c < CHUNKS)
        def _():
            sl = pl.ds(0, C)
            pltpu.make_async_copy(x_hbm.at[sl], xv[p], sem_in[p]).wait()
            pltpu.make_async_copy(y_hbm.at[sl], yv[p], sem_in[p]).wait()
            pltpu.make_async_copy(z_hbm.at[sl], zv[p], sem_in[p]).wait()
            pltpu.make_async_copy(b_hbm.at[sl], bv[p], sem_in[p]).wait()

    def compute(k, p):
        c = w + NW * k

        @pl.when(c < CHUNKS)
        def _():
            def step(i, carry):
                s = pl.ds(i * 16, 16)
                xf = xv[p][s]
                yf = yv[p][s]
                zf = zv[p][s]
                bi = bv[p][s]
                gx = ((xf * 0.5 + 0.5) * 128.0).astype(jnp.int32)
                gy = ((yf * 0.5 + 0.5) * 128.0).astype(jnp.int32)
                gz = ((zf * 0.5 + 0.5) * 128.0).astype(jnp.int32)
                gx = jnp.minimum(jnp.maximum(gx, 0), 127)
                gy = jnp.minimum(jnp.maximum(gy, 0), 127)
                gz = jnp.minimum(jnp.maximum(gz, 0), 127)
                f = ((bi * 128 + gx) * 128 + gy) * 128 + gz
                idxv[p][s] = f
                return carry

            lax.fori_loop(0, C // 16, step, 0)

    def fire_gather(k, p):
        c = w + NW * k

        @pl.when(c < CHUNKS)
        def _():
            pltpu.async_copy(tab_hbm.at[idxv[p]], gv[p], sem_g[p])

    def wait_out(k, p):
        c = w + NW * k

        @pl.when(c < CHUNKS)
        def _():
            pltpu.make_async_copy(gv[p], out_hbm.at[pl.ds(0, C)], sem_o[p]).wait()

    def drain_gather(k, p):
        c = w + NW * k

        @pl.when(c < CHUNKS)
        def _():
            pltpu.make_async_copy(tab_hbm.at[idxv[p]], gv[p], sem_g[p]).wait()
            pltpu.async_copy(gv[p], out_hbm.at[pl.ds(c * C, C)], sem_o[p])

    start_in(0, 0)

    def outer(it, carry):
        for p in (0, 1):
            k = 2 * it + p
            wait_in(k, p)
            compute(k, p)

            @pl.when(k > 1)
            def _():
                wait_out(k - 2, p)

            fire_gather(k, p)

            @pl.when(k > 0)
            def _():
                drain_gather(k - 1, p ^ 1)

            start_in(k + 1, p ^ 1)
        return carry

    lax.fori_loop(0, ITERS // 2, outer, 0)
    drain_gather(ITERS - 1, (ITERS - 1) & 1)
    wait_out(ITERS - 2, (ITERS - 2) & 1)
    wait_out(ITERS - 1, (ITERS - 1) & 1)


def kernel(pts, bidx, occ_grid_per_batch, resolution):
    x = pts[:, 0]
    y = pts[:, 1]
    z = pts[:, 2]
    tab = occ_grid_per_batch.reshape(-1)

    mesh = plsc.VectorSubcoreMesh(core_axis_name="c", subcore_axis_name="s")
    out = pl.kernel(
        _body,
        out_type=jax.ShapeDtypeStruct((N,), jnp.bool_),
        mesh=mesh,
        scratch_types=[
            [pltpu.VMEM((C,), jnp.float32)] * 2,
            [pltpu.VMEM((C,), jnp.float32)] * 2,
            [pltpu.VMEM((C,), jnp.float32)] * 2,
            [pltpu.VMEM((C,), jnp.int32)] * 2,
            [pltpu.VMEM((C,), jnp.int32)] * 2,
            [pltpu.VMEM((C,), jnp.bool_)] * 2,
            [pltpu.SemaphoreType.DMA] * 2,
            [pltpu.SemaphoreType.DMA] * 2,
            [pltpu.SemaphoreType.DMA] * 2,
        ],
    )(x, y, z, bidx, tab)
    return out
